# baseline (speedup 1.0000x reference)
import functools
import math

import jax
import jax.numpy as jnp
from jax import lax
from jax.experimental import pallas as pl
from jax.experimental.pallas import tpu as pltpu
from jax.experimental.pallas import tpu_sc as plsc

D_MODEL = 64
SCALE = math.sqrt(D_MODEL)
CHUNK = 128
NBUF = 5


_DBLK = 400


def _detile_body(a_ref, b_ref, out_ref):
    out_ref[...] = jnp.concatenate([a_ref[...], b_ref[...]], axis=1)


def _detile(table, V):
    H = V // 2
    assert H % _DBLK == 0
    nb = H // _DBLK
    return pl.pallas_call(
        _detile_body,
        grid=(nb,),
        in_specs=[
            pl.BlockSpec((_DBLK, D_MODEL), lambda i: (i, 0)),
            pl.BlockSpec((_DBLK, D_MODEL), lambda i: (i + nb, 0)),
        ],
        out_specs=pl.BlockSpec((_DBLK, 2 * D_MODEL), lambda i: (i, 0)),
        out_shape=jax.ShapeDtypeStruct((H, 2 * D_MODEL), jnp.float32),
    )(table, table)


@functools.lru_cache(maxsize=None)
def _build(B, V, n_rows, n_cols):
    info = plsc.get_sparse_core_info()
    NW = info.num_cores * info.num_subcores
    NC = info.num_cores
    assert B % (NW * CHUNK) == 0
    b_per_w = B // NW
    n_chunks = b_per_w // CHUNK
    assert n_chunks % NBUF == 0

    mesh = plsc.VectorSubcoreMesh(core_axis_name="c", subcore_axis_name="s")

    scratch = [pltpu.VMEM((n_chunks, CHUNK), jnp.int32)]
    scratch += [pltpu.VMEM((CHUNK, D_MODEL), jnp.float32) for _ in range(NBUF)]
    scratch += [pltpu.SemaphoreType.DMA for _ in range(2 * NBUF)]

    @functools.partial(
        pl.kernel,
        mesh=mesh,
        compiler_params=pltpu.CompilerParams(use_tc_tiling_on_sc=False),
        out_type=jax.ShapeDtypeStruct((B, D_MODEL), jnp.float32),
        scratch_types=scratch,
    )
    def emb_kernel(idx_hbm, table_hbm, out_hbm, idx_v, *bufs_and_sems):
        bufs = bufs_and_sems[:NBUF]
        sem_g = bufs_and_sems[NBUF:2 * NBUF]
        sem_s = bufs_and_sems[2 * NBUF:]
        wid = lax.axis_index("s") * NC + lax.axis_index("c")
        base = wid * b_per_w

        pltpu.sync_copy(idx_hbm.at[wid], idx_v)

        def gather_start(c, b):
            pltpu.make_async_copy(
                table_hbm.at[idx_v.at[c]], bufs[b], sem_g[b]).start()

        def gather_wait(b):
            pltpu.make_async_copy(
                table_hbm.at[idx_v.at[0]], bufs[b], sem_g[b]).wait()

        def store_start(c, b):
            pltpu.make_async_copy(
                bufs[b], out_hbm.at[pl.ds(base + c * CHUNK, CHUNK)],
                sem_s[b]).start()

        def store_wait(b):
            pltpu.make_async_copy(
                bufs[b], out_hbm.at[pl.ds(base, CHUNK)], sem_s[b]).wait()

        for b in range(NBUF - 1):
            gather_start(b, b)

        def outer_body(o, _):
            for b in range(NBUF):
                c = o + b
                gather_wait(b)

                @plsc.parallel_loop(0, CHUNK, step=1, unroll=4)
                def scale_row(r):
                    for k in range(D_MODEL // 16):
                        sl = (r, pl.ds(k * 16, 16))
                        bufs[b][sl] = bufs[b][sl] * SCALE

                store_start(c, b)
                bp = (b - 1) % NBUF
                p = c + NBUF - 1

                @pl.when((c >= 1) & (p < n_chunks))
                def _():
                    store_wait(bp)

                @pl.when(p < n_chunks)
                def _():
                    gather_start(p, bp)

            return 0

        lax.fori_loop(0, n_chunks // NBUF, lambda i, _: outer_body(i * NBUF, 0), 0)

        for b in range(NBUF):
            store_wait(b)

    def run(x, table):
        xi = x.reshape(-1).astype(jnp.int32)
        half = (xi >= V // 2).astype(jnp.int32)
        i64 = 2 * (xi - half * (V // 2)) + half
        idx3d = i64.reshape(NW, n_chunks, CHUNK)
        table_lin = _detile(table, V).reshape(V, D_MODEL)
        out = emb_kernel(idx3d, table_lin)
        return out.reshape(n_rows, n_cols, D_MODEL)

    return run


def kernel(x, table):
    n_rows, n_cols = x.shape
    V = table.shape[0]
    return _build(n_rows * n_cols, V, n_rows, n_cols)(x, table)


# speedup vs baseline: 1.7425x; 1.7425x over previous
import functools
import math

import jax
import jax.numpy as jnp
from jax import lax
from jax.experimental import pallas as pl
from jax.experimental.pallas import tpu as pltpu
from jax.experimental.pallas import tpu_sc as plsc

D_MODEL = 64
SCALE = math.sqrt(D_MODEL)
CHUNK = 128
NBUF = 5


_DBLK = 4000


def _detile_body(a_ref, b_ref, out_ref):
    out_ref[...] = jnp.concatenate([a_ref[...], b_ref[...]], axis=1)


def _detile(table, V):
    H = V // 2
    assert H % _DBLK == 0
    nb = H // _DBLK
    return pl.pallas_call(
        _detile_body,
        grid=(nb,),
        in_specs=[
            pl.BlockSpec((_DBLK, D_MODEL), lambda i: (i, 0)),
            pl.BlockSpec((_DBLK, D_MODEL), lambda i: (i + nb, 0)),
        ],
        out_specs=pl.BlockSpec((_DBLK, 2 * D_MODEL), lambda i: (i, 0)),
        out_shape=jax.ShapeDtypeStruct((H, 2 * D_MODEL), jnp.float32),
    )(table, table)


@functools.lru_cache(maxsize=None)
def _build(B, V, n_rows, n_cols):
    info = plsc.get_sparse_core_info()
    NW = info.num_cores * info.num_subcores
    NC = info.num_cores
    assert B % (NW * CHUNK) == 0
    b_per_w = B // NW
    n_chunks = b_per_w // CHUNK
    assert n_chunks % NBUF == 0

    mesh = plsc.VectorSubcoreMesh(core_axis_name="c", subcore_axis_name="s")

    scratch = [pltpu.VMEM((n_chunks, CHUNK), jnp.int32)]
    scratch += [pltpu.VMEM((CHUNK, D_MODEL), jnp.float32) for _ in range(NBUF)]
    scratch += [pltpu.SemaphoreType.DMA for _ in range(2 * NBUF)]

    @functools.partial(
        pl.kernel,
        mesh=mesh,
        compiler_params=pltpu.CompilerParams(use_tc_tiling_on_sc=False),
        out_type=jax.ShapeDtypeStruct((B, D_MODEL), jnp.float32),
        scratch_types=scratch,
    )
    def emb_kernel(idx_hbm, table_hbm, out_hbm, idx_v, *bufs_and_sems):
        bufs = bufs_and_sems[:NBUF]
        sem_g = bufs_and_sems[NBUF:2 * NBUF]
        sem_s = bufs_and_sems[2 * NBUF:]
        wid = lax.axis_index("s") * NC + lax.axis_index("c")
        base = wid * b_per_w

        pltpu.sync_copy(idx_hbm.at[wid], idx_v)

        def gather_start(c, b):
            pltpu.make_async_copy(
                table_hbm.at[idx_v.at[c]], bufs[b], sem_g[b]).start()

        def gather_wait(b):
            pltpu.make_async_copy(
                table_hbm.at[idx_v.at[0]], bufs[b], sem_g[b]).wait()

        def store_start(c, b):
            pltpu.make_async_copy(
                bufs[b], out_hbm.at[pl.ds(base + c * CHUNK, CHUNK)],
                sem_s[b]).start()

        def store_wait(b):
            pltpu.make_async_copy(
                bufs[b], out_hbm.at[pl.ds(base, CHUNK)], sem_s[b]).wait()

        for b in range(NBUF - 1):
            gather_start(b, b)

        def outer_body(o, _):
            for b in range(NBUF):
                c = o + b
                gather_wait(b)

                @plsc.parallel_loop(0, CHUNK, step=1, unroll=4)
                def scale_row(r):
                    for k in range(D_MODEL // 16):
                        sl = (r, pl.ds(k * 16, 16))
                        bufs[b][sl] = bufs[b][sl] * SCALE

                store_start(c, b)
                bp = (b - 1) % NBUF
                p = c + NBUF - 1

                @pl.when((c >= 1) & (p < n_chunks))
                def _():
                    store_wait(bp)

                @pl.when(p < n_chunks)
                def _():
                    gather_start(p, bp)

            return 0

        lax.fori_loop(0, n_chunks // NBUF, lambda i, _: outer_body(i * NBUF, 0), 0)

        for b in range(NBUF):
            store_wait(b)

    def run(x, table):
        xi = x.reshape(-1).astype(jnp.int32)
        half = (xi >= V // 2).astype(jnp.int32)
        i64 = 2 * (xi - half * (V // 2)) + half
        idx3d = i64.reshape(NW, n_chunks, CHUNK)
        table_lin = _detile(table, V).reshape(V, D_MODEL)
        out = emb_kernel(idx3d, table_lin)
        return out.reshape(n_rows, n_cols, D_MODEL)

    return run


def kernel(x, table):
    n_rows, n_cols = x.shape
    V = table.shape[0]
    return _build(n_rows * n_cols, V, n_rows, n_cols)(x, table)


# speedup vs baseline: 3.0345x; 1.7414x over previous
import functools
import math

import jax
import jax.numpy as jnp
from jax import lax
from jax.experimental import pallas as pl
from jax.experimental.pallas import tpu as pltpu
from jax.experimental.pallas import tpu_sc as plsc

D_MODEL = 64
SCALE = math.sqrt(D_MODEL)
CHUNK = 128
NBUF = 5


_DBLK = 4096


def _detile_body(a_ref, b_ref, out_ref):
    out_ref[...] = jnp.concatenate([a_ref[...].T, b_ref[...].T], axis=1)


def _detile(table_t, V):
    nb = (V + 2 * _DBLK - 1) // (2 * _DBLK)
    last_blk = (V - 1) // _DBLK
    return pl.pallas_call(
        _detile_body,
        grid=(nb,),
        in_specs=[
            pl.BlockSpec((D_MODEL, _DBLK), lambda i: (0, 2 * i)),
            pl.BlockSpec((D_MODEL, _DBLK),
                         lambda i: (0, jnp.minimum(2 * i + 1, last_blk))),
        ],
        out_specs=pl.BlockSpec((_DBLK, 2 * D_MODEL), lambda i: (i, 0)),
        out_shape=jax.ShapeDtypeStruct((nb * _DBLK, 2 * D_MODEL),
                                       jnp.float32),
    )(table_t, table_t)


@functools.lru_cache(maxsize=None)
def _build(B, V, n_rows, n_cols):
    info = plsc.get_sparse_core_info()
    NW = info.num_cores * info.num_subcores
    NC = info.num_cores
    assert B % (NW * CHUNK) == 0
    b_per_w = B // NW
    n_chunks = b_per_w // CHUNK
    assert n_chunks % NBUF == 0

    mesh = plsc.VectorSubcoreMesh(core_axis_name="c", subcore_axis_name="s")

    scratch = [pltpu.VMEM((n_chunks, CHUNK), jnp.int32)]
    scratch += [pltpu.VMEM((CHUNK, D_MODEL), jnp.float32) for _ in range(NBUF)]
    scratch += [pltpu.SemaphoreType.DMA for _ in range(2 * NBUF)]

    @functools.partial(
        pl.kernel,
        mesh=mesh,
        compiler_params=pltpu.CompilerParams(use_tc_tiling_on_sc=False),
        out_type=jax.ShapeDtypeStruct((B, D_MODEL), jnp.float32),
        scratch_types=scratch,
    )
    def emb_kernel(idx_hbm, table_hbm, out_hbm, idx_v, *bufs_and_sems):
        bufs = bufs_and_sems[:NBUF]
        sem_g = bufs_and_sems[NBUF:2 * NBUF]
        sem_s = bufs_and_sems[2 * NBUF:]
        wid = lax.axis_index("s") * NC + lax.axis_index("c")
        base = wid * b_per_w

        pltpu.sync_copy(idx_hbm.at[wid], idx_v)

        def gather_start(c, b):
            pltpu.make_async_copy(
                table_hbm.at[idx_v.at[c]], bufs[b], sem_g[b]).start()

        def gather_wait(b):
            pltpu.make_async_copy(
                table_hbm.at[idx_v.at[0]], bufs[b], sem_g[b]).wait()

        def store_start(c, b):
            pltpu.make_async_copy(
                bufs[b], out_hbm.at[pl.ds(base + c * CHUNK, CHUNK)],
                sem_s[b]).start()

        def store_wait(b):
            pltpu.make_async_copy(
                bufs[b], out_hbm.at[pl.ds(base, CHUNK)], sem_s[b]).wait()

        for b in range(NBUF - 1):
            gather_start(b, b)

        def outer_body(o, _):
            for b in range(NBUF):
                c = o + b
                gather_wait(b)

                @plsc.parallel_loop(0, CHUNK, step=1, unroll=4)
                def scale_row(r):
                    for k in range(D_MODEL // 16):
                        sl = (r, pl.ds(k * 16, 16))
                        bufs[b][sl] = bufs[b][sl] * SCALE

                store_start(c, b)
                bp = (b - 1) % NBUF
                p = c + NBUF - 1

                @pl.when((c >= 1) & (p < n_chunks))
                def _():
                    store_wait(bp)

                @pl.when(p < n_chunks)
                def _():
                    gather_start(p, bp)

            return 0

        lax.fori_loop(0, n_chunks // NBUF, lambda i, _: outer_body(i * NBUF, 0), 0)

        for b in range(NBUF):
            store_wait(b)

    def run(x, table):
        xi = x.reshape(-1).astype(jnp.int32)
        g = xi >> 12
        r = xi & (_DBLK - 1)
        i64 = 2 * ((g >> 1) * _DBLK + r) + (g & 1)
        idx3d = i64.reshape(NW, n_chunks, CHUNK)
        table2 = _detile(table.T, V)
        table_lin = table2.reshape(2 * table2.shape[0], D_MODEL)
        out = emb_kernel(idx3d, table_lin)
        return out.reshape(n_rows, n_cols, D_MODEL)

    return run


def kernel(x, table):
    n_rows, n_cols = x.shape
    V = table.shape[0]
    return _build(n_rows * n_cols, V, n_rows, n_cols)(x, table)
